# Initial kernel scaffold; baseline (speedup 1.0000x reference)
import jax
import jax.numpy as jnp
from jax.experimental import pallas as pl


def kernel(x, edge_attr, edge_index, batch, proj_node_w, proj_node_b, proj_edge_w, proj_edge_b, gin0_w1, gin0_b1, gin0_w2, gin0_b2, gin1_w1, gin1_b1, gin1_w2, gin1_b2, gin2_w1, gin2_b1, gin2_w2, gin2_b2):
    raise NotImplementedError("write your pallas kernel here")


# speedup vs baseline: 2.1421x; 2.1421x over previous
import functools

import jax
import jax.numpy as jnp
from jax import lax
from jax.experimental import pallas as pl
from jax.experimental.pallas import tpu as pltpu
from jax.experimental.pallas import tpu_sc as plsc

N_NODES = 10000
N_EDGES = 320000
D_IN = 128
D_E = 16
D_H = 300
HALF = 150
HPAD = 160
N_GRAPHS = 16
DEPTH = 3

NC, NS, NLANE = 2, 16, 16
EB = 80
EDGES_PER_TILE = N_EDGES // (NC * NS)
NBLK = EDGES_PER_TILE // EB
NPAD = 10000
ROWS_PER_TILE = NPAD // NS
ZROWS = 25


def _sc_body(h_hbm, e_hbm, src_hbm, dst_hbm, out_hbm,
             sidx, didx, gbuf, ebuf, zbuf, agg, sem):
    c = lax.axis_index("c")
    s = lax.axis_index("s")
    tile = c * NS + s

    def zrow(r, carry):
        for j in range(HPAD // NLANE):
            zbuf[r, pl.ds(j * NLANE, NLANE)] = jnp.zeros((NLANE,), jnp.float32)
        return carry

    lax.fori_loop(0, ZROWS, zrow, 0)
    row0 = s * ROWS_PER_TILE

    def zcp(i, carry):
        pltpu.sync_copy(zbuf, agg.at[pl.ds(row0 + i * ZROWS, ZROWS)])
        return carry

    lax.fori_loop(0, ROWS_PER_TILE // ZROWS, zcp, 0)
    plsc.subcore_barrier()

    ebase = tile * EDGES_PER_TILE

    def blk(b, carry):
        base = ebase + b * EB
        pltpu.sync_copy(src_hbm.at[pl.ds(base, EB)], sidx)
        pltpu.sync_copy(dst_hbm.at[pl.ds(base, EB)], didx)
        pltpu.async_copy(h_hbm.at[sidx], gbuf, sem).wait()
        pltpu.sync_copy(e_hbm.at[pl.ds(base, EB)], ebuf)

        def rw(r, rc):
            for j in range(HPAD // NLANE):
                sl = pl.ds(j * NLANE, NLANE)
                ebuf[r, sl] = jnp.maximum(ebuf[r, sl] + gbuf[r, sl], 0.0)
            return rc

        lax.fori_loop(0, EB, rw, 0)
        pltpu.sync_copy(ebuf, agg.at[didx], add=True)
        return carry

    lax.fori_loop(0, NBLK, blk, 0)
    plsc.subcore_barrier()
    pltpu.sync_copy(agg.at[pl.ds(row0, ROWS_PER_TILE)],
                    out_hbm.at[c, pl.ds(row0, ROWS_PER_TILE)])


_sc_half = functools.partial(
    pl.kernel,
    out_type=jax.ShapeDtypeStruct((NC, NPAD, HPAD), jnp.float32),
    mesh=plsc.VectorSubcoreMesh(core_axis_name="c", subcore_axis_name="s"),
    compiler_params=pltpu.CompilerParams(use_tc_tiling_on_sc=False),
    scratch_types=[
        pltpu.VMEM((EB,), jnp.int32),
        pltpu.VMEM((EB,), jnp.int32),
        pltpu.VMEM((EB, HPAD), jnp.float32),
        pltpu.VMEM((EB, HPAD), jnp.float32),
        pltpu.VMEM((ZROWS, HPAD), jnp.float32),
        pltpu.VMEM_SHARED((NPAD, HPAD), jnp.float32),
        pltpu.SemaphoreType.DMA,
    ],
)(_sc_body)


def _pad_halves(h):
    rows = h.shape[0]
    z = jnp.zeros((rows, HPAD - HALF), jnp.float32)
    return (jnp.concatenate([h[:, :HALF], z], axis=1),
            jnp.concatenate([h[:, HALF:], z], axis=1))


def _node_proj_body(x_ref, wT_ref, b_ref, hL_ref, hR_ref):
    h = jnp.dot(x_ref[...], wT_ref[...], preferred_element_type=jnp.float32)
    h = jnp.maximum(h + b_ref[...], 0.0)
    hL, hR = _pad_halves(h)
    hL_ref[...] = hL
    hR_ref[...] = hR


def _edge_proj_body(ea_ref, wT_ref, b_ref, eL_ref, eR_ref):
    e = jnp.dot(ea_ref[...], wT_ref[...], preferred_element_type=jnp.float32)
    e = e + b_ref[...]
    eL, eR = _pad_halves(e)
    eL_ref[...] = eL
    eR_ref[...] = eR


def _mlp_body(hL_ref, hR_ref, aL_ref, aR_ref, w1T_ref, b1_ref, w2T_ref,
              b2_ref, hLo_ref, hRo_ref, *, trailing_relu):
    zL = hL_ref[:, :HALF] + aL_ref[0, :, :HALF] + aL_ref[1, :, :HALF]
    zR = hR_ref[:, :HALF] + aR_ref[0, :, :HALF] + aR_ref[1, :, :HALF]
    z = jnp.concatenate([zL, zR], axis=1)
    t = jnp.dot(z, w1T_ref[...], preferred_element_type=jnp.float32)
    t = jnp.maximum(t + b1_ref[...], 0.0)
    h2 = jnp.dot(t, w2T_ref[...], preferred_element_type=jnp.float32)
    h2 = h2 + b2_ref[...]
    if trailing_relu:
        h2 = jnp.maximum(h2, 0.0)
    hL, hR = _pad_halves(h2)
    hLo_ref[...] = hL
    hRo_ref[...] = hR


def _final_body(batch_ref, hL_ref, hR_ref, aL_ref, aR_ref, w1T_ref, b1_ref,
                w2T_ref, b2_ref, out_ref, *, rows):
    i = pl.program_id(0)
    zL = hL_ref[:, :HALF] + aL_ref[0, :, :HALF] + aL_ref[1, :, :HALF]
    zR = hR_ref[:, :HALF] + aR_ref[0, :, :HALF] + aR_ref[1, :, :HALF]
    z = jnp.concatenate([zL, zR], axis=1)
    t = jnp.dot(z, w1T_ref[...], preferred_element_type=jnp.float32)
    t = jnp.maximum(t + b1_ref[...], 0.0)
    h2 = jnp.dot(t, w2T_ref[...], preferred_element_type=jnp.float32)
    h2 = h2 + b2_ref[...]
    batch = batch_ref[...]
    gs = lax.broadcasted_iota(jnp.int32, (N_GRAPHS, 1), 0)
    counts = jnp.sum((batch <= gs).astype(jnp.int32), axis=1, keepdims=True)
    idx = (counts - 1) % N_NODES
    glob = lax.broadcasted_iota(jnp.int32, (1, rows), 1) + i * rows
    onehot = (glob == idx).astype(jnp.float32)

    @pl.when(i == 0)
    def _():
        out_ref[...] = jnp.zeros_like(out_ref)

    out_ref[...] += jnp.dot(onehot, h2, preferred_element_type=jnp.float32)


def _node_proj(x, wT, b):
    return pl.pallas_call(
        _node_proj_body,
        out_shape=[jax.ShapeDtypeStruct((N_NODES, HPAD), jnp.float32)] * 2,
    )(x, wT, b)


def _edge_proj(ea, wT, b):
    epb = 8000
    grid = (N_EDGES // epb,)
    return pl.pallas_call(
        _edge_proj_body,
        grid=grid,
        in_specs=[
            pl.BlockSpec((epb, D_E), lambda i: (i, 0)),
            pl.BlockSpec((D_E, D_H), lambda i: (0, 0)),
            pl.BlockSpec((1, D_H), lambda i: (0, 0)),
        ],
        out_specs=[pl.BlockSpec((epb, HPAD), lambda i: (i, 0))] * 2,
        out_shape=[jax.ShapeDtypeStruct((N_EDGES, HPAD), jnp.float32)] * 2,
    )(ea, wT, b)


def _mlp(hL, hR, aL, aR, w1T, b1, w2T, b2, trailing_relu):
    rb = 2000
    grid = (N_NODES // rb,)
    half_spec = pl.BlockSpec((rb, HPAD), lambda i: (i, 0))
    agg_spec = pl.BlockSpec((NC, rb, HPAD), lambda i: (0, i, 0))
    w_spec = pl.BlockSpec((D_H, D_H), lambda i: (0, 0))
    b_spec = pl.BlockSpec((1, D_H), lambda i: (0, 0))
    return pl.pallas_call(
        functools.partial(_mlp_body, trailing_relu=trailing_relu),
        grid=grid,
        in_specs=[half_spec, half_spec, agg_spec, agg_spec,
                  w_spec, b_spec, w_spec, b_spec],
        out_specs=[half_spec] * 2,
        out_shape=[jax.ShapeDtypeStruct((N_NODES, HPAD), jnp.float32)] * 2,
    )(hL, hR, aL, aR, w1T, b1, w2T, b2)


def _final(batch2d, hL, hR, aL, aR, w1T, b1, w2T, b2):
    rb = 2000
    grid = (N_NODES // rb,)
    half_spec = pl.BlockSpec((rb, HPAD), lambda i: (i, 0))
    agg_spec = pl.BlockSpec((NC, rb, HPAD), lambda i: (0, i, 0))
    w_spec = pl.BlockSpec((D_H, D_H), lambda i: (0, 0))
    b_spec = pl.BlockSpec((1, D_H), lambda i: (0, 0))
    return pl.pallas_call(
        functools.partial(_final_body, rows=rb),
        grid=grid,
        in_specs=[pl.BlockSpec((1, N_NODES), lambda i: (0, 0)),
                  half_spec, half_spec, agg_spec, agg_spec,
                  w_spec, b_spec, w_spec, b_spec],
        out_specs=pl.BlockSpec((N_GRAPHS, D_H), lambda i: (0, 0)),
        out_shape=jax.ShapeDtypeStruct((N_GRAPHS, D_H), jnp.float32),
    )(batch2d, hL, hR, aL, aR, w1T, b1, w2T, b2)


def kernel(x, edge_attr, edge_index, batch, proj_node_w, proj_node_b,
           proj_edge_w, proj_edge_b, gin0_w1, gin0_b1, gin0_w2, gin0_b2,
           gin1_w1, gin1_b1, gin1_w2, gin1_b2, gin2_w1, gin2_b1, gin2_w2,
           gin2_b2):
    src = edge_index[0].astype(jnp.int32)
    dst = edge_index[1].astype(jnp.int32)
    batch2d = batch.astype(jnp.int32).reshape(1, N_NODES)

    hL, hR = _node_proj(x, proj_node_w.T, proj_node_b.reshape(1, D_H))
    eL, eR = _edge_proj(edge_attr, proj_edge_w.T, proj_edge_b.reshape(1, D_H))

    gin = [(gin0_w1, gin0_b1, gin0_w2, gin0_b2),
           (gin1_w1, gin1_b1, gin1_w2, gin1_b2),
           (gin2_w1, gin2_b1, gin2_w2, gin2_b2)]

    for i in range(DEPTH - 1):
        w1, b1, w2, b2 = gin[i]
        aL = _sc_half(hL, eL, src, dst)
        aR = _sc_half(hR, eR, src, dst)
        hL, hR = _mlp(hL, hR, aL, aR, w1.T, b1.reshape(1, D_H),
                      w2.T, b2.reshape(1, D_H), trailing_relu=True)

    w1, b1, w2, b2 = gin[DEPTH - 1]
    aL = _sc_half(hL, eL, src, dst)
    aR = _sc_half(hR, eR, src, dst)
    return _final(batch2d, hL, hR, aL, aR, w1.T, b1.reshape(1, D_H),
                  w2.T, b2.reshape(1, D_H))
